# Initial kernel scaffold; baseline (speedup 1.0000x reference)
import jax
import jax.numpy as jnp
from jax.experimental import pallas as pl


def kernel(x, edge_index):
    raise NotImplementedError("write your pallas kernel here")


# speedup vs baseline: 3.2562x; 3.2562x over previous
import jax
import jax.numpy as jnp
from jax.experimental import pallas as pl

N = 4096
K = 8


def _concat_kernel(x_ref, pe_ref, o_ref):
    o_ref[...] = jnp.concatenate([x_ref[...], pe_ref[...]], axis=1)


def kernel(x, edge_index):
    rows = edge_index[0]
    cols = edge_index[1]
    deg = jax.ops.segment_sum(jnp.ones(rows.shape, jnp.float32), rows, num_segments=N)
    deg_inv = jnp.where(deg > 0, 1.0 / deg, 0.0)

    M = jnp.zeros((N, N), jnp.float32).at[rows, cols].add(deg_inv[rows])
    P2 = M @ M
    P3 = M @ P2
    P4 = M @ P3

    w = deg_inv[rows]
    d1 = jax.ops.segment_sum(w * (rows == cols), rows, num_segments=N)
    ds = [d1]
    for P in (M, P2, P3, P4):
        g = P[cols, rows]
        ds.append(jax.ops.segment_sum(w * g, rows, num_segments=N))
    d6 = jnp.einsum("ij,ji->i", P3, P3)
    d7 = jnp.einsum("ij,ji->i", P3, P4)
    d8 = jnp.einsum("ij,ji->i", P4, P4)
    pe = jnp.stack(ds + [d6, d7, d8], axis=1)

    return pl.pallas_call(
        _concat_kernel,
        out_shape=jax.ShapeDtypeStruct((N, 72), jnp.float32),
    )(x, pe)


# speedup vs baseline: 4.9663x; 1.5252x over previous
import functools

import jax
import jax.numpy as jnp
from jax.experimental import pallas as pl

N = 4096
BM = 1024
BR = 128


def _mm_kernel(a_ref, b_ref, o_ref):
    o_ref[...] = jnp.dot(
        a_ref[...], b_ref[...], preferred_element_type=jnp.float32
    ).astype(jnp.bfloat16)


@jax.jit
def _mm(a, b):
    grid = (N // BM, N // BM)
    return pl.pallas_call(
        _mm_kernel,
        grid=grid,
        in_specs=[
            pl.BlockSpec((BM, N), lambda i, j: (i, 0)),
            pl.BlockSpec((N, BM), lambda i, j: (0, j)),
        ],
        out_specs=pl.BlockSpec((BM, BM), lambda i, j: (i, j)),
        out_shape=jax.ShapeDtypeStruct((N, N), jnp.bfloat16),
    )(a, b)


def _rowdot_kernel(p3r_ref, p4r_ref, p3c_ref, p4c_ref, x_ref, d5_ref, o_ref):
    eye = jnp.eye(BR, dtype=jnp.float32)
    def ddot(r, c):
        prod = jnp.dot(r[...], c[...], preferred_element_type=jnp.float32)
        return jnp.sum(prod * eye, axis=1, keepdims=True)
    d6 = ddot(p3r_ref, p3c_ref)
    d7 = ddot(p3r_ref, p4c_ref)
    d8 = ddot(p4r_ref, p4c_ref)
    o_ref[...] = jnp.concatenate([x_ref[...], d5_ref[...], d6, d7, d8], axis=1)


@jax.jit
def _rowdot_assemble(p3, p4, x, d15):
    grid = (N // BR,)
    return pl.pallas_call(
        _rowdot_kernel,
        grid=grid,
        in_specs=[
            pl.BlockSpec((BR, N), lambda i: (i, 0)),
            pl.BlockSpec((BR, N), lambda i: (i, 0)),
            pl.BlockSpec((N, BR), lambda i: (0, i)),
            pl.BlockSpec((N, BR), lambda i: (0, i)),
            pl.BlockSpec((BR, 64), lambda i: (i, 0)),
            pl.BlockSpec((BR, 5), lambda i: (i, 0)),
        ],
        out_specs=pl.BlockSpec((BR, 72), lambda i: (i, 0)),
        out_shape=jax.ShapeDtypeStruct((N, 72), jnp.float32),
    )(p3, p4, p3, p4, x, d15)


def kernel(x, edge_index):
    rows = edge_index[0]
    cols = edge_index[1]
    deg = jax.ops.segment_sum(jnp.ones(rows.shape, jnp.float32), rows, num_segments=N)
    deg_inv = jnp.where(deg > 0, 1.0 / deg, 0.0)

    M = jnp.zeros((N, N), jnp.float32).at[rows, cols].add(deg_inv[rows])
    Mb = M.astype(jnp.bfloat16)
    P2 = _mm(Mb, Mb)
    P3 = _mm(Mb, P2)
    P4 = _mm(Mb, P3)

    w = deg_inv[rows]
    d1 = jax.ops.segment_sum(w * (rows == cols), rows, num_segments=N)
    ds = [d1]
    for P in (M, P2, P3, P4):
        g = P[cols, rows].astype(jnp.float32)
        ds.append(jax.ops.segment_sum(w * g, rows, num_segments=N))
    d15 = jnp.stack(ds, axis=1)

    return _rowdot_assemble(P3, P4, x, d15)
